# Initial kernel scaffold; baseline (speedup 1.0000x reference)
import jax
import jax.numpy as jnp
from jax.experimental import pallas as pl


def kernel(qlen, klen, emb):
    raise NotImplementedError("write your pallas kernel here")


# speedup vs baseline: 27.6298x; 27.6298x over previous
import math

import jax
import jax.numpy as jnp
from jax.experimental import pallas as pl
from jax.experimental.pallas import tpu as pltpu

HEADS = 16
NUM_BUCKETS = 32
MAX_DISTANCE = 128
QLEN = 2048
KLEN = 2048
TW = 4096
BI = 256
SUB = 8


def _bias_kernel(off_ref, embT_ref, out_ref):
    ib = pl.program_id(1)
    offset = off_ref[0]

    t = jax.lax.broadcasted_iota(jnp.int32, (1, TW), 1)
    d = t - (QLEN - 1) - offset
    n = -d
    half = NUM_BUCKETS // 2
    ret = jnp.where(n < 0, half, 0)
    na = jnp.abs(n)
    max_exact = half // 2
    nf = jnp.maximum(na, 1).astype(jnp.float32)
    val_large = max_exact + (
        jnp.log(nf / max_exact)
        / math.log(MAX_DISTANCE / max_exact)
        * (half - max_exact)
    ).astype(jnp.int32)
    val_large = jnp.minimum(val_large, half - 1)
    bucket = ret + jnp.where(na < max_exact, na, val_large)

    V = jnp.zeros((1, TW), jnp.float32)
    for b in range(NUM_BUCKETS):
        V = V + jnp.where(bucket == b, embT_ref[0, 0:1, b : b + 1], 0.0)

    si = jax.lax.broadcasted_iota(jnp.int32, (SUB, TW), 0)
    shifts = (SUB - 1) - si
    Vs = jnp.broadcast_to(V, (SUB, TW))
    bit = 1
    while bit < SUB:
        rolled = pltpu.roll(Vs, TW - bit, 1)
        Vs = jnp.where((shifts & bit) != 0, rolled, Vs)
        bit *= 2

    def body(g, _):
        i0 = ib * BI + g * SUB
        a = (QLEN - 1) - (SUB - 1) - i0
        W = pltpu.roll(Vs, (TW - a) % TW, 1)
        out_ref[0, pl.ds(g * SUB, SUB), :] = W[:, :KLEN]
        return 0

    jax.lax.fori_loop(0, BI // SUB, body, 0)


def kernel(qlen, klen, emb):
    offset = (jnp.asarray(klen) - jnp.asarray(qlen)).astype(jnp.int32)
    off = jnp.reshape(offset, (1,))
    embT = emb.T.reshape(HEADS, 1, NUM_BUCKETS)

    out = pl.pallas_call(
        _bias_kernel,
        grid=(HEADS, QLEN // BI),
        in_specs=[
            pl.BlockSpec(memory_space=pltpu.SMEM),
            pl.BlockSpec((1, 1, NUM_BUCKETS), lambda h, ib: (h, 0, 0)),
        ],
        out_specs=pl.BlockSpec((1, BI, KLEN), lambda h, ib: (h, ib, 0)),
        out_shape=jax.ShapeDtypeStruct((HEADS, QLEN, KLEN), jnp.float32),
    )(off, embT)
    return out


# speedup vs baseline: 125.1566x; 4.5298x over previous
import math

import jax
import jax.numpy as jnp
from jax.experimental import pallas as pl
from jax.experimental.pallas import tpu as pltpu

HEADS = 16
NUM_BUCKETS = 32
MAX_DISTANCE = 128
QLEN = 2048
KLEN = 2048
TW = 4096
BI = 256
SUB = 8
NG = BI // SUB


def _bias_kernel(off_ref, embT_ref, out_ref, vs_ref):
    ib = pl.program_id(1)
    offset = off_ref[0]

    @pl.when(ib == 0)
    def _build_table():
        t = jax.lax.broadcasted_iota(jnp.int32, (1, TW), 1)
        d = t - (QLEN - 1) - offset
        n = -d
        half = NUM_BUCKETS // 2
        ret = jnp.where(n < 0, half, 0)
        na = jnp.abs(n)
        max_exact = half // 2
        nf = jnp.maximum(na, 1).astype(jnp.float32)
        val_large = max_exact + (
            jnp.log(nf / max_exact)
            / math.log(MAX_DISTANCE / max_exact)
            * (half - max_exact)
        ).astype(jnp.int32)
        val_large = jnp.minimum(val_large, half - 1)
        bucket = ret + jnp.where(na < max_exact, na, val_large)

        V = jnp.zeros((1, TW), jnp.float32)
        for b in range(NUM_BUCKETS):
            V = V + jnp.where(bucket == b, embT_ref[0, 0:1, b : b + 1], 0.0)

        si = jax.lax.broadcasted_iota(jnp.int32, (SUB, TW), 0)
        shifts = (SUB - 1) - si
        Vs = jnp.broadcast_to(V, (SUB, TW))
        bit = 1
        while bit < SUB:
            rolled = pltpu.roll(Vs, TW - bit, 1)
            Vs = jnp.where((shifts & bit) != 0, rolled, Vs)
            bit *= 2
        vs_ref[...] = Vs

    i0 = ib * BI
    a_last = (QLEN - 1) - (SUB - 1) - i0 - SUB * (NG - 1)
    w = pltpu.roll(vs_ref[...], (TW - a_last) % TW, 1)
    for g in range(NG):
        off_g = SUB * (NG - 1 - g)
        out_ref[0, SUB * g : SUB * (g + 1), :] = w[:, off_g : off_g + KLEN]


def kernel(qlen, klen, emb):
    offset = (jnp.asarray(klen) - jnp.asarray(qlen)).astype(jnp.int32)
    off = jnp.reshape(offset, (1,))
    embT = emb.T.reshape(HEADS, 1, NUM_BUCKETS)

    out = pl.pallas_call(
        _bias_kernel,
        grid=(HEADS, QLEN // BI),
        in_specs=[
            pl.BlockSpec(memory_space=pltpu.SMEM),
            pl.BlockSpec((1, 1, NUM_BUCKETS), lambda h, ib: (h, 0, 0)),
        ],
        out_specs=pl.BlockSpec((1, BI, KLEN), lambda h, ib: (h, ib, 0)),
        out_shape=jax.ShapeDtypeStruct((HEADS, QLEN, KLEN), jnp.float32),
        scratch_shapes=[pltpu.VMEM((SUB, TW), jnp.float32)],
    )(off, embT)
    return out


# speedup vs baseline: 189.8530x; 1.5169x over previous
import math

import jax
import jax.numpy as jnp
from jax.experimental import pallas as pl
from jax.experimental.pallas import tpu as pltpu

HEADS = 16
NUM_BUCKETS = 32
MAX_DISTANCE = 128
QLEN = 2048
KLEN = 2048
TW = 4096
SUB = 8
GROUP = 128


def _bias_kernel(off_ref, embT_ref, out_ref, vs_ref):
    offset = off_ref[0]

    lane = jax.lax.broadcasted_iota(jnp.int32, (SUB, TW), 1)
    sub = jax.lax.broadcasted_iota(jnp.int32, (SUB, TW), 0)
    t = lane + (SUB - 1) - sub
    d = t - (QLEN - 1) - offset
    n = -d
    half = NUM_BUCKETS // 2
    ret = jnp.where(n < 0, half, 0)
    na = jnp.abs(n)
    max_exact = half // 2
    nf = jnp.maximum(na, 1).astype(jnp.float32)
    val_large = max_exact + (
        jnp.log(nf / max_exact)
        / math.log(MAX_DISTANCE / max_exact)
        * (half - max_exact)
    ).astype(jnp.int32)
    val_large = jnp.minimum(val_large, half - 1)
    bucket = ret + jnp.where(na < max_exact, na, val_large)

    Vs8 = jnp.zeros((SUB, TW), jnp.float32)
    for b in range(NUM_BUCKETS):
        Vs8 = Vs8 + jnp.where(bucket == b, embT_ref[0, 0:1, b : b + 1], 0.0)

    for a in range(GROUP // SUB):
        shift = SUB * (GROUP // SUB - 1 - a)
        vs_ref[SUB * a : SUB * (a + 1), :] = pltpu.roll(Vs8, (TW - shift) % TW, 1)

    for g in range(QLEN // GROUP):
        s = (QLEN - 1) - (GROUP - 1) - GROUP * g
        out_ref[0, GROUP * g : GROUP * (g + 1), :] = vs_ref[:, s : s + KLEN]


def kernel(qlen, klen, emb):
    offset = (jnp.asarray(klen) - jnp.asarray(qlen)).astype(jnp.int32)
    off = jnp.reshape(offset, (1,))
    embT = emb.T.reshape(HEADS, 1, NUM_BUCKETS)

    out = pl.pallas_call(
        _bias_kernel,
        grid=(HEADS,),
        in_specs=[
            pl.BlockSpec(memory_space=pltpu.SMEM),
            pl.BlockSpec((1, 1, NUM_BUCKETS), lambda h: (h, 0, 0)),
        ],
        out_specs=pl.BlockSpec((1, QLEN, KLEN), lambda h: (h, 0, 0)),
        out_shape=jax.ShapeDtypeStruct((HEADS, QLEN, KLEN), jnp.float32),
        scratch_shapes=[pltpu.VMEM((GROUP, TW), jnp.float32)],
    )(off, embT)
    return out
